# baseline (speedup 1.0000x reference)
import functools

import jax
import jax.numpy as jnp
from jax import lax
from jax.experimental import pallas as pl
from jax.experimental.pallas import tpu as pltpu

B, N, D, DO = 256, 64, 16, 16
BB = 8


def _tc_body(v_ref, f_ref, a_ref, w1_ref, w2_ref, w3_ref, b_ref, o_ref):
    ii = lax.broadcasted_iota(jnp.int32, (N, N), 0)
    jj = lax.broadcasted_iota(jnp.int32, (N, N), 1)
    eye = (ii == jj).astype(jnp.float32)
    i128 = lax.broadcasted_iota(jnp.int32, (N, 2 * N), 0)
    l128 = lax.broadcasted_iota(jnp.int32, (N, 2 * N), 1)
    dup = ((l128 // 2) == i128).astype(jnp.float32)

    w1 = w1_ref[...]
    w2 = w2_ref[...]
    w3 = w3_ref[...]
    bias = b_ref[...]
    hi = jax.lax.Precision.HIGHEST
    for t in range(BB):
        f = f_ref[t]
        a = a_ref[t]
        v = v_ref[t]
        g = jnp.dot(f, w2, precision=hi)
        base = jnp.dot(f, w1, precision=hi) + bias
        s = jnp.dot(a, g, precision=hi)
        adup = jnp.dot(a, dup, precision=hi)
        dcon = jnp.dot(adup * v, w3, precision=hi)
        aii = jnp.sum(a * eye, axis=1, keepdims=True)
        selfdiff = jnp.dot(v * dup, w3, precision=hi)
        out = base + s + dcon - aii * (g + selfdiff)
        o_ref[t] = jnp.maximum(out, 0.0)


@jax.jit
def _run_tc(v, f, a, w1, w2, w3rep, bs):
    grid = (B // BB,)
    out = pl.pallas_call(
        _tc_body,
        grid=grid,
        in_specs=[
            pl.BlockSpec((BB, N, 2 * N), lambda p: (p, 0, 0)),
            pl.BlockSpec((BB, N, D), lambda p: (p, 0, 0)),
            pl.BlockSpec((BB, N, N), lambda p: (p, 0, 0)),
            pl.BlockSpec((D, DO), lambda p: (0, 0)),
            pl.BlockSpec((D, DO), lambda p: (0, 0)),
            pl.BlockSpec((2 * N, DO), lambda p: (0, 0)),
            pl.BlockSpec((1, DO), lambda p: (0, 0)),
        ],
        out_specs=pl.BlockSpec((BB, N, DO), lambda p: (p, 0, 0)),
        out_shape=jax.ShapeDtypeStruct((B, N, DO), jnp.float32),
        compiler_params=pltpu.CompilerParams(
            dimension_semantics=("arbitrary",),
        ),
    )(v, f, a, w1, w2, w3rep, bs)
    return out


def kernel(diff_vecs, agent_features, A, W, b):
    v = diff_vecs.reshape(B, N, 2 * N)
    w1 = (N - 1.0) * W[:, :D].T
    w2 = W[:, D:2 * D].T
    w3rep = jnp.tile(W[:, 2 * D:].T, (N, 1))
    bs = ((N - 1.0) * b).reshape(1, DO)
    out = _run_tc(v, agent_features, A, w1, w2, w3rep, bs)
    return (diff_vecs, out)
